# baseline (speedup 1.0000x reference)
import functools

import jax
import jax.numpy as jnp
from jax import lax
from jax.experimental import pallas as pl
from jax.experimental.pallas import tpu as pltpu
from jax.experimental.pallas import tpu_sc as plsc

_NC = 2
_NS = 16
_NW = _NC * _NS
_L = 16


def _sc_diag_gather(flat_hbm, y_hbm, oy_hbm, idx_v, oy_v, sem, *, C, b_per_w):
    wid = lax.axis_index("s") * _NC + lax.axis_index("c")
    base = wid * b_per_w
    n_seg = b_per_w // 128
    for k in range(n_seg):
        pltpu.sync_copy(y_hbm.at[pl.ds(base + k * 128, 128)], idx_v.at[k])
    for k in range(n_seg):
        for j in range(128 // _L):
            row0 = base + k * 128 + j * _L
            rows = lax.iota(jnp.int32, _L) + row0
            idx_v[k, pl.ds(j * _L, _L)] = idx_v[k, pl.ds(j * _L, _L)] + rows * C
    copies = [
        pltpu.async_copy(flat_hbm.at[idx_v.at[k]], oy_v.at[pl.ds(k * 128, 128)], sem)
        for k in range(n_seg)
    ]
    for c in copies:
        c.wait()
    pltpu.sync_copy(oy_v, oy_hbm.at[pl.ds(base, b_per_w)])


def _tc_body(x_ref, oy_ref, o_ref, *, C):
    x = x_ref[...]
    a = oy_ref[...] - 1.0
    hinge = jnp.maximum(x - a[:, None], 0.0)
    o_ref[...] = (jnp.sum(hinge, axis=1) - 1.0) * (1.0 / C)


def kernel(output, y):
    B, C = output.shape
    b_per_w = B // _NW
    n_seg = b_per_w // 128
    flat = output.reshape(-1)

    mesh = plsc.VectorSubcoreMesh(core_axis_name="c", subcore_axis_name="s")
    oy = pl.kernel(
        functools.partial(_sc_diag_gather, C=C, b_per_w=b_per_w),
        out_type=jax.ShapeDtypeStruct((B,), jnp.float32),
        mesh=mesh,
        scratch_types=[
            pltpu.VMEM((n_seg, 128), jnp.int32),
            pltpu.VMEM((b_per_w,), jnp.float32),
            pltpu.SemaphoreType.DMA,
        ],
    )(flat, y)

    R = 256
    return pl.pallas_call(
        functools.partial(_tc_body, C=C),
        grid=(B // R,),
        in_specs=[
            pl.BlockSpec((R, C), lambda i: (i, 0)),
            pl.BlockSpec((R,), lambda i: (i,)),
        ],
        out_specs=pl.BlockSpec((R,), lambda i: (i,)),
        out_shape=jax.ShapeDtypeStruct((B,), jnp.float32),
    )(output, oy)


# speedup vs baseline: 1.1673x; 1.1673x over previous
import functools

import jax
import jax.numpy as jnp
from jax import lax
from jax.experimental import pallas as pl
from jax.experimental.pallas import tpu as pltpu
from jax.experimental.pallas import tpu_sc as plsc

_NC = 2
_NS = 16
_NW = _NC * _NS
_L = 16


def _sc_loss(x_hbm, y_hbm, o_hbm, y_v, buf, loss_v, a_buf, sem0, sem1, *, B, C):
    b_per_w = B // _NW
    n_chunks = b_per_w // _L
    wid = lax.axis_index("s") * _NC + lax.axis_index("c")
    base = wid * b_per_w
    lanes = lax.iota(jnp.int32, _L)
    n_full = C // _L
    rem = C % _L
    sems = (sem0, sem1)

    pltpu.sync_copy(y_hbm.at[pl.ds(base, b_per_w)], y_v)

    def start(g, b):
        pltpu.async_copy(x_hbm.at[pl.ds(base + g * _L, _L), :], buf.at[b], sems[b])

    start(0, 0)
    start(1, 1)

    def do_pair(p, _):
        for b in (0, 1):
            g = 2 * p + b
            pltpu.make_async_copy(
                x_hbm.at[pl.ds(0, _L), :], buf.at[b], sems[b]).wait()
            y16 = y_v[pl.ds(g * _L, _L)]
            diag = plsc.load_gather(buf.at[b], [lanes, y16])
            a_buf[...] = diag - 1.0

            def row(i, lvec):
                bcast = plsc.load_gather(a_buf, [jnp.full((_L,), i, jnp.int32)])
                acc = jnp.zeros((_L,), jnp.float32)
                for t in range(n_full - 1 if rem else n_full):
                    v = buf[b, i, pl.ds(t * _L, _L)]
                    acc = acc + jnp.maximum(v - bcast, 0.0)
                if rem:
                    v = buf[b, i, pl.ds((n_full - 1) * _L, _L)]
                    acc = acc + jnp.maximum(v - bcast, 0.0)
                    v = buf[b, i, pl.ds(C - _L, _L)]
                    r = jnp.maximum(v - bcast, 0.0)
                    r = jnp.where(lanes >= (_L - rem), r, 0.0)
                    acc = acc + r
                rowsum = jnp.sum(acc)
                return jnp.where(lanes == i, (rowsum - 1.0) * (1.0 / C), lvec)

            lvec = lax.fori_loop(0, _L, row, jnp.zeros((_L,), jnp.float32))
            loss_v[pl.ds(g * _L, _L)] = lvec

            @pl.when(g + 2 < n_chunks)
            def _():
                pltpu.async_copy(
                    x_hbm.at[pl.ds(base + (g + 2) * _L, _L), :], buf.at[b], sems[b])

        return None

    lax.fori_loop(0, n_chunks // 2, do_pair, None)
    pltpu.sync_copy(loss_v, o_hbm.at[pl.ds(base, b_per_w)])


def kernel(output, y):
    B, C = output.shape
    b_per_w = B // _NW
    mesh = plsc.VectorSubcoreMesh(core_axis_name="c", subcore_axis_name="s")
    return pl.kernel(
        functools.partial(_sc_loss, B=B, C=C),
        out_type=jax.ShapeDtypeStruct((B,), jnp.float32),
        mesh=mesh,
        compiler_params=pltpu.CompilerParams(
            use_tc_tiling_on_sc=False, needs_layout_passes=False),
        scratch_types=[
            pltpu.VMEM((b_per_w,), jnp.int32),
            pltpu.VMEM((2, _L, C), jnp.float32),
            pltpu.VMEM((b_per_w,), jnp.float32),
            pltpu.VMEM((_L,), jnp.float32),
            pltpu.SemaphoreType.DMA,
            pltpu.SemaphoreType.DMA,
        ],
    )(output, y)


# speedup vs baseline: 2.1237x; 1.8193x over previous
import functools

import jax
import jax.numpy as jnp
from jax.experimental import pallas as pl
from jax.experimental.pallas import tpu as pltpu


def _body(x_ref, y_ref, o_ref, *, C):
    x = x_ref[...]
    yv = y_ref[...]
    R = x.shape[0]
    col = jax.lax.broadcasted_iota(jnp.int32, (R, C), 1)
    onehot = col == yv[:, None]
    oy = jnp.sum(jnp.where(onehot, x, 0.0), axis=1, keepdims=True)
    hinge = jnp.maximum(x - oy + 1.0, 0.0)
    o_ref[...] = (jnp.sum(hinge, axis=1) - 1.0) * (1.0 / C)


def kernel(output, y):
    B, C = output.shape
    R = 512
    grid = (B // R,)
    return pl.pallas_call(
        functools.partial(_body, C=C),
        grid=grid,
        in_specs=[
            pl.BlockSpec((R, C), lambda i: (i, 0)),
            pl.BlockSpec((R,), lambda i: (i,)),
        ],
        out_specs=pl.BlockSpec((R,), lambda i: (i,)),
        out_shape=jax.ShapeDtypeStruct((B,), jnp.float32),
    )(output, y)


# speedup vs baseline: 2.3404x; 1.1021x over previous
import functools

import jax
import jax.numpy as jnp
from jax.experimental import pallas as pl
from jax.experimental.pallas import tpu as pltpu


def _body(x_ref, y_ref, o_ref, *, C):
    x = x_ref[...]
    yv = y_ref[...]
    R = x.shape[0]
    col = jax.lax.broadcasted_iota(jnp.int32, (R, C), 1)
    onehot = col == yv[:, None]
    oy = jnp.sum(jnp.where(onehot, x, 0.0), axis=1, keepdims=True)
    hinge = jnp.maximum(x - oy + 1.0, 0.0)
    o_ref[...] = (jnp.sum(hinge, axis=1) - 1.0) * (1.0 / C)


def kernel(output, y):
    B, C = output.shape
    R = 1024
    grid = (B // R,)
    return pl.pallas_call(
        functools.partial(_body, C=C),
        grid=grid,
        in_specs=[
            pl.BlockSpec((R, C), lambda i: (i, 0)),
            pl.BlockSpec((R,), lambda i: (i,)),
        ],
        out_specs=pl.BlockSpec((R,), lambda i: (i,)),
        out_shape=jax.ShapeDtypeStruct((B,), jnp.float32),
    )(output, y)


# speedup vs baseline: 2.4445x; 1.0445x over previous
import functools

import jax
import jax.numpy as jnp
from jax.experimental import pallas as pl
from jax.experimental.pallas import tpu as pltpu


def _body(x_ref, y_ref, o_ref, *, C):
    x = x_ref[...]
    yv = y_ref[...]
    R = x.shape[0]
    col = jax.lax.broadcasted_iota(jnp.int32, (R, C), 1)
    onehot = col == yv[:, None]
    oy = jnp.sum(jnp.where(onehot, x, 0.0), axis=1, keepdims=True)
    hinge = jnp.maximum(x - oy + 1.0, 0.0)
    o_ref[...] = (jnp.sum(hinge, axis=1) - 1.0) * (1.0 / C)


def kernel(output, y):
    B, C = output.shape
    R = 2048
    grid = (B // R,)
    return pl.pallas_call(
        functools.partial(_body, C=C),
        grid=grid,
        in_specs=[
            pl.BlockSpec((R, C), lambda i: (i, 0)),
            pl.BlockSpec((R,), lambda i: (i,)),
        ],
        out_specs=pl.BlockSpec((R,), lambda i: (i,)),
        out_shape=jax.ShapeDtypeStruct((B,), jnp.float32),
    )(output, y)
